# baseline (speedup 1.0000x reference)
import functools

import jax
import jax.numpy as jnp
from jax.experimental import pallas as pl
from jax.experimental.pallas import tpu as pltpu
from jax.experimental.pallas import tpu_sc as plsc

_NUM_EXPERTS = 8
_TOP_K = 2
_BLOCK_T = 2048
_NW = 32
_LANES = 16


def _logits_kernel(x_ref, w_ref, lt_ref):
    logits = jnp.dot(x_ref[...], w_ref[...],
                     preferred_element_type=jnp.float32)
    lt_ref[...] = logits.T


def _sc_route_body(lt_hbm, wt_hbm, it_hbm, ps_hbm, fs_hbm,
                   ltc, wloc, iloc, sfp, sff, *, chunk, ngroups):
    c = jax.lax.axis_index("c")
    s = jax.lax.axis_index("s")
    wid = s * 2 + c
    base = wid * chunk
    pltpu.sync_copy(lt_hbm.at[:, pl.ds(base, chunk)], ltc)

    zeros = jnp.zeros((_LANES,), jnp.float32)
    one = jnp.ones((_LANES,), jnp.float32)

    def group(g, carry):
        pacc = carry[:8]
        facc = carry[8:]
        sl = pl.ds(g * _LANES, _LANES)
        l = [ltc[e, sl] for e in range(8)]
        m1 = l[0]
        i1 = jnp.zeros((_LANES,), jnp.int32)
        m2 = jnp.full((_LANES,), -jnp.inf, jnp.float32)
        i2 = jnp.zeros((_LANES,), jnp.int32)
        for e in range(1, 8):
            ev = jnp.full((_LANES,), e, jnp.int32)
            gt1 = l[e] > m1
            gt2 = l[e] > m2
            sa = jnp.where(gt2, l[e], m2)
            sb = jnp.where(gt2, ev, i2)
            m2 = jnp.where(gt1, m1, sa)
            i2 = jnp.where(gt1, i1, sb)
            m1 = jnp.where(gt1, l[e], m1)
            i1 = jnp.where(gt1, ev, i1)
        e21 = jnp.exp(m2 - m1)
        w1 = 1.0 / (1.0 + e21)
        wloc[0, sl] = w1
        wloc[1, sl] = 1.0 - w1
        iloc[0, sl] = i1
        iloc[1, sl] = i2
        z = [jnp.exp(l[e] - m1) for e in range(8)]
        ssum = z[0]
        for e in range(1, 8):
            ssum = ssum + z[e]
        inv = 1.0 / ssum
        pacc = tuple(pacc[e] + z[e] * inv for e in range(8))
        facc = tuple(
            facc[e] + jnp.where(i1 == jnp.full((_LANES,), e, jnp.int32),
                                one, zeros)
            for e in range(8))
        return pacc + facc

    init = tuple(zeros for _ in range(16))
    res = jax.lax.fori_loop(0, ngroups, group, init)
    for e in range(8):
        sfp[e, :] = res[e]
        sff[e, :] = res[8 + e]
    pltpu.sync_copy(wloc, wt_hbm.at[:, pl.ds(base, chunk)])
    pltpu.sync_copy(iloc, it_hbm.at[:, pl.ds(base, chunk)])
    pltpu.sync_copy(sfp, ps_hbm.at[wid])
    pltpu.sync_copy(sff, fs_hbm.at[wid])


def _aux_kernel(ps_ref, fs_ref, aux_ref, *, n_tokens):
    pe = jnp.sum(ps_ref[...], axis=(0, 2))
    fe = jnp.sum(fs_ref[...], axis=(0, 2))
    scale = 1.0 / (n_tokens * n_tokens)
    aux_ref[0, 0] = _NUM_EXPERTS * scale * jnp.sum(pe * fe)


def kernel(x, W):
    B, S, D = x.shape
    E = W.shape[1]
    n = B * S
    x2 = x.reshape(n, D)
    num_blocks = n // _BLOCK_T
    chunk = n // _NW
    ngroups = chunk // _LANES

    lt = pl.pallas_call(
        _logits_kernel,
        grid=(num_blocks,),
        in_specs=[
            pl.BlockSpec((_BLOCK_T, D), lambda i: (i, 0)),
            pl.BlockSpec((D, E), lambda i: (0, 0)),
        ],
        out_specs=pl.BlockSpec((E, _BLOCK_T), lambda i: (0, i)),
        out_shape=jax.ShapeDtypeStruct((E, n), jnp.float32),
    )(x2, W)

    mesh = plsc.VectorSubcoreMesh(core_axis_name="c", subcore_axis_name="s")
    wt, it, ps, fs = pl.kernel(
        functools.partial(_sc_route_body, chunk=chunk, ngroups=ngroups),
        out_type=[
            jax.ShapeDtypeStruct((_TOP_K, n), jnp.float32),
            jax.ShapeDtypeStruct((_TOP_K, n), jnp.int32),
            jax.ShapeDtypeStruct((_NW, E, _LANES), jnp.float32),
            jax.ShapeDtypeStruct((_NW, E, _LANES), jnp.float32),
        ],
        mesh=mesh,
        scratch_types=[
            pltpu.VMEM((E, chunk), jnp.float32),
            pltpu.VMEM((_TOP_K, chunk), jnp.float32),
            pltpu.VMEM((_TOP_K, chunk), jnp.int32),
            pltpu.VMEM((E, _LANES), jnp.float32),
            pltpu.VMEM((E, _LANES), jnp.float32),
        ],
    )(lt)

    aux = pl.pallas_call(
        functools.partial(_aux_kernel, n_tokens=n),
        in_specs=[
            pl.BlockSpec((_NW, E, _LANES), lambda: (0, 0, 0)),
            pl.BlockSpec((_NW, E, _LANES), lambda: (0, 0, 0)),
        ],
        out_specs=pl.BlockSpec(memory_space=pltpu.SMEM),
        out_shape=jax.ShapeDtypeStruct((1, 1), jnp.float32),
    )(ps, fs)

    return (wt.T.reshape(B, S, _TOP_K),
            it.T.reshape(B, S, _TOP_K).astype(jnp.int64),
            aux[0, 0])


# speedup vs baseline: 1.3297x; 1.3297x over previous
import functools

import jax
import jax.numpy as jnp
from jax.experimental import pallas as pl
from jax.experimental.pallas import tpu as pltpu

_NUM_EXPERTS = 8
_TOP_K = 2
_BLOCK_T = 1024
_NBUF = 4


def _route_block(lt, wt_ref, it_ref, f_acc, p_acc, blk):
    E, T = lt.shape
    si = jax.lax.broadcasted_iota(jnp.int32, (E, T), 0)

    m1 = jnp.max(lt, axis=0, keepdims=True)
    is_max = lt == m1
    idx1 = jnp.min(jnp.where(is_max, si, E), axis=0, keepdims=True)
    nmax = jnp.sum(is_max.astype(jnp.int32), axis=0, keepdims=True)
    mx = jnp.max(jnp.where(is_max, -jnp.inf, lt), axis=0, keepdims=True)
    m2 = jnp.where(nmax > 1, m1, mx)
    idx2 = jnp.min(jnp.where((lt == m2) & (si != idx1), si, E),
                   axis=0, keepdims=True)

    e21 = jnp.exp(m2 - m1)
    w1 = 1.0 / (1.0 + e21)
    w2 = 1.0 - w1
    wt_ref[:, blk, :] = jnp.concatenate([w1, w2], axis=0)
    it_ref[:, blk, :] = jnp.concatenate([idx1, idx2], axis=0)

    z = jnp.exp(lt - m1)
    p_acc[...] += z * (1.0 / jnp.sum(z, axis=0, keepdims=True))
    f_acc[...] += (si == idx1).astype(jnp.float32)


def _router_kernel(x_hbm, w_ref, wt_ref, it_ref, aux_ref,
                   xbuf, f_acc, p_acc, sems, *, n_tokens, num_blocks):
    f_acc[...] = jnp.zeros_like(f_acc)
    p_acc[...] = jnp.zeros_like(p_acc)
    E = _NUM_EXPERTS
    T = _BLOCK_T

    for b in range(_NBUF):
        pltpu.make_async_copy(x_hbm.at[pl.ds(b * T, T), :],
                              xbuf.at[b], sems.at[b]).start()

    def outer(j, carry):
        for b in range(_NBUF):
            i = j * _NBUF + b
            pltpu.make_async_copy(x_hbm.at[pl.ds(i * T, T), :],
                                  xbuf.at[b], sems.at[b]).wait()
            logits = jnp.dot(xbuf[b], w_ref[...],
                             preferred_element_type=jnp.float32)
            _route_block(logits.T, wt_ref, it_ref, f_acc, p_acc, i)
            nxt = i + _NBUF

            @pl.when(nxt < num_blocks)
            def _():
                pltpu.make_async_copy(x_hbm.at[pl.ds(nxt * T, T), :],
                                      xbuf.at[b], sems.at[b]).start()
        return carry

    jax.lax.fori_loop(0, num_blocks // _NBUF, outer, 0)

    scale = 1.0 / (n_tokens * n_tokens)
    fe = jnp.sum(f_acc[...], axis=1)
    pe = jnp.sum(p_acc[...], axis=1)
    aux_ref[0, 0] = E * scale * jnp.sum(fe * pe)


def kernel(x, W):
    B, S, D = x.shape
    E = W.shape[1]
    n = B * S
    x2 = x.reshape(n, D)
    num_blocks = n // _BLOCK_T

    wt, it, aux = pl.pallas_call(
        functools.partial(_router_kernel, n_tokens=n, num_blocks=num_blocks),
        in_specs=[
            pl.BlockSpec(memory_space=pl.ANY),
            pl.BlockSpec(memory_space=pltpu.VMEM),
        ],
        out_specs=[
            pl.BlockSpec(memory_space=pltpu.VMEM),
            pl.BlockSpec(memory_space=pltpu.VMEM),
            pl.BlockSpec(memory_space=pltpu.SMEM),
        ],
        out_shape=[
            jax.ShapeDtypeStruct((_TOP_K, num_blocks, _BLOCK_T), jnp.float32),
            jax.ShapeDtypeStruct((_TOP_K, num_blocks, _BLOCK_T), jnp.int32),
            jax.ShapeDtypeStruct((1, 1), jnp.float32),
        ],
        scratch_shapes=[
            pltpu.VMEM((_NBUF, _BLOCK_T, 768), jnp.float32),
            pltpu.VMEM((_NUM_EXPERTS, _BLOCK_T), jnp.float32),
            pltpu.VMEM((_NUM_EXPERTS, _BLOCK_T), jnp.float32),
            pltpu.SemaphoreType.DMA((_NBUF,)),
        ],
    )(x2, W)
    return (wt.reshape(_TOP_K, n).T.reshape(B, S, _TOP_K),
            it.reshape(_TOP_K, n).T.reshape(B, S, _TOP_K).astype(jnp.int64),
            aux[0, 0])


# speedup vs baseline: 1.3380x; 1.0063x over previous
import functools

import jax
import jax.numpy as jnp
from jax.experimental import pallas as pl
from jax.experimental.pallas import tpu as pltpu

_NUM_EXPERTS = 8
_TOP_K = 2
_BLOCK_T = 2048
_NBUF = 4


def _route_block(lt, wt_ref, it_ref, f_acc, p_acc, blk):
    E, T = lt.shape
    si = jax.lax.broadcasted_iota(jnp.int32, (E, T), 0)

    m1 = jnp.max(lt, axis=0, keepdims=True)
    is_max = lt == m1
    idx1 = jnp.min(jnp.where(is_max, si, E), axis=0, keepdims=True)
    nmax = jnp.sum(is_max.astype(jnp.int32), axis=0, keepdims=True)
    mx = jnp.max(jnp.where(is_max, -jnp.inf, lt), axis=0, keepdims=True)
    m2 = jnp.where(nmax > 1, m1, mx)
    idx2 = jnp.min(jnp.where((lt == m2) & (si != idx1), si, E),
                   axis=0, keepdims=True)

    e21 = jnp.exp(m2 - m1)
    w1 = 1.0 / (1.0 + e21)
    w2 = 1.0 - w1
    wt_ref[:, blk, :] = jnp.concatenate([w1, w2], axis=0)
    it_ref[:, blk, :] = jnp.concatenate([idx1, idx2], axis=0)

    z = jnp.exp(lt - m1)
    p_acc[...] += z * (1.0 / jnp.sum(z, axis=0, keepdims=True))
    f_acc[...] += (si == idx1).astype(jnp.float32)


def _router_kernel(x_hbm, w_ref, wt_ref, it_ref, aux_ref,
                   xbuf, f_acc, p_acc, sems, *, n_tokens, num_blocks):
    f_acc[...] = jnp.zeros_like(f_acc)
    p_acc[...] = jnp.zeros_like(p_acc)
    E = _NUM_EXPERTS
    T = _BLOCK_T

    for b in range(_NBUF):
        pltpu.make_async_copy(x_hbm.at[pl.ds(b * T, T), :],
                              xbuf.at[b], sems.at[b]).start()

    def outer(j, carry):
        for b in range(_NBUF):
            i = j * _NBUF + b
            pltpu.make_async_copy(x_hbm.at[pl.ds(i * T, T), :],
                                  xbuf.at[b], sems.at[b]).wait()
            logits = jnp.dot(xbuf[b], w_ref[...],
                             preferred_element_type=jnp.float32)
            _route_block(logits.T, wt_ref, it_ref, f_acc, p_acc, i)
            nxt = i + _NBUF

            @pl.when(nxt < num_blocks)
            def _():
                pltpu.make_async_copy(x_hbm.at[pl.ds(nxt * T, T), :],
                                      xbuf.at[b], sems.at[b]).start()
        return carry

    jax.lax.fori_loop(0, num_blocks // _NBUF, outer, 0)

    scale = 1.0 / (n_tokens * n_tokens)
    fe = jnp.sum(f_acc[...], axis=1)
    pe = jnp.sum(p_acc[...], axis=1)
    aux_ref[0, 0] = E * scale * jnp.sum(fe * pe)


def kernel(x, W):
    B, S, D = x.shape
    E = W.shape[1]
    n = B * S
    x2 = x.reshape(n, D)
    num_blocks = n // _BLOCK_T

    wt, it, aux = pl.pallas_call(
        functools.partial(_router_kernel, n_tokens=n, num_blocks=num_blocks),
        in_specs=[
            pl.BlockSpec(memory_space=pl.ANY),
            pl.BlockSpec(memory_space=pltpu.VMEM),
        ],
        out_specs=[
            pl.BlockSpec(memory_space=pltpu.VMEM),
            pl.BlockSpec(memory_space=pltpu.VMEM),
            pl.BlockSpec(memory_space=pltpu.SMEM),
        ],
        out_shape=[
            jax.ShapeDtypeStruct((_TOP_K, num_blocks, _BLOCK_T), jnp.float32),
            jax.ShapeDtypeStruct((_TOP_K, num_blocks, _BLOCK_T), jnp.int32),
            jax.ShapeDtypeStruct((1, 1), jnp.float32),
        ],
        scratch_shapes=[
            pltpu.VMEM((_NBUF, _BLOCK_T, 768), jnp.float32),
            pltpu.VMEM((_NUM_EXPERTS, _BLOCK_T), jnp.float32),
            pltpu.VMEM((_NUM_EXPERTS, _BLOCK_T), jnp.float32),
            pltpu.SemaphoreType.DMA((_NBUF,)),
        ],
    )(x2, W)
    return (wt.reshape(_TOP_K, n).T.reshape(B, S, _TOP_K),
            it.reshape(_TOP_K, n).T.reshape(B, S, _TOP_K).astype(jnp.int64),
            aux[0, 0])


# speedup vs baseline: 1.5829x; 1.1830x over previous
import functools

import jax
import jax.numpy as jnp
from jax.experimental import pallas as pl
from jax.experimental.pallas import tpu as pltpu

_NUM_EXPERTS = 8
_TOP_K = 2
_BLOCK_T = 4096


def _router_kernel(x_ref, w_ref, weights_ref, idx_ref, aux_ref,
                   f_acc, p_acc, *, n_tokens, num_blocks):
    i = pl.program_id(0)

    @pl.when(i == 0)
    def _init():
        f_acc[...] = jnp.zeros_like(f_acc)
        p_acc[...] = jnp.zeros_like(p_acc)

    logits = jnp.dot(x_ref[...], w_ref[...],
                     preferred_element_type=jnp.float32)
    lt = logits.T
    E, T = lt.shape
    si = jax.lax.broadcasted_iota(jnp.int32, (E, T), 0)

    m1 = jnp.max(lt, axis=0, keepdims=True)
    idx1 = jnp.min(jnp.where(lt == m1, si, E), axis=0, keepdims=True)
    masked = jnp.where(si == idx1, -jnp.inf, lt)
    m2 = jnp.max(masked, axis=0, keepdims=True)
    idx2 = jnp.min(jnp.where(masked == m2, si, E), axis=0, keepdims=True)

    e21 = jnp.exp(m2 - m1)
    w1 = 1.0 / (1.0 + e21)
    w2 = 1.0 - w1
    weights_ref[...] = jnp.concatenate([w1, w2], axis=0)
    idx_ref[...] = jnp.concatenate([idx1, idx2], axis=0)

    z = jnp.exp(lt - m1)
    p_acc[...] += z / jnp.sum(z, axis=0, keepdims=True)
    f_acc[...] += (si == idx1).astype(jnp.float32)

    @pl.when(i == num_blocks - 1)
    def _finish():
        scale = 1.0 / (n_tokens * n_tokens)
        fe = jnp.sum(f_acc[...], axis=1)
        pe = jnp.sum(p_acc[...], axis=1)
        aux_ref[0, 0] = E * scale * jnp.sum(fe * pe)


def kernel(x, W):
    B, S, D = x.shape
    E = W.shape[1]
    n = B * S
    x2 = x.reshape(n, D)
    num_blocks = n // _BLOCK_T

    grid_spec = pltpu.PrefetchScalarGridSpec(
        num_scalar_prefetch=0,
        grid=(num_blocks,),
        in_specs=[
            pl.BlockSpec((_BLOCK_T, D), lambda i: (i, 0)),
            pl.BlockSpec((D, E), lambda i: (0, 0)),
        ],
        out_specs=[
            pl.BlockSpec((_TOP_K, _BLOCK_T), lambda i: (0, i)),
            pl.BlockSpec((_TOP_K, _BLOCK_T), lambda i: (0, i)),
            pl.BlockSpec((1, 1), lambda i: (0, 0), memory_space=pltpu.SMEM),
        ],
        scratch_shapes=[
            pltpu.VMEM((E, _BLOCK_T), jnp.float32),
            pltpu.VMEM((E, _BLOCK_T), jnp.float32),
        ],
    )
    weights_t, idx_t, aux = pl.pallas_call(
        functools.partial(_router_kernel, n_tokens=n, num_blocks=num_blocks),
        grid_spec=grid_spec,
        out_shape=[
            jax.ShapeDtypeStruct((_TOP_K, n), jnp.float32),
            jax.ShapeDtypeStruct((_TOP_K, n), jnp.int32),
            jax.ShapeDtypeStruct((1, 1), jnp.float32),
        ],
    )(x2, W)
    return (weights_t.T.reshape(B, S, _TOP_K),
            idx_t.T.reshape(B, S, _TOP_K).astype(jnp.int64),
            aux[0, 0])
